# baseline (speedup 1.0000x reference)
import jax
import jax.numpy as jnp
from jax.experimental import pallas as pl
from jax.experimental.pallas import tpu as pltpu

S = 7
B = 2
C = 20
N = B * 5 + C
CELLS_PER_ROW = 8
LANES = N * CELLS_PER_ROW
LOBJ = 5.0
LNOBJ = 0.5

ROWS_PER_BLOCK = 1792


def _loss_slab(pT, tT):
    inv_s = jnp.float32(1.0 / S)

    def corners(v, c0):
        xy = v[c0:c0 + 2] * inv_s
        half = v[c0 + 2:c0 + 4] * 0.5
        return xy - half, xy + half

    l0, r0 = corners(pT, 0)
    l1, r1 = corners(pT, 5)
    lb, rb = corners(tT, 0)
    area_b = tT[2:3] * tT[3:4]

    def iou(la, ra, area_a):
        lt = jnp.maximum(la, lb)
        rb_ = jnp.minimum(ra, rb)
        wh = jnp.maximum(rb_ - lt, 0.0)
        inter = wh[0:1] * wh[1:2]
        return inter / (area_a + area_b - inter + 1e-10)

    i0 = iou(l0, r0, pT[2:3] * pT[3:4])
    i1 = iou(l1, r1, pT[7:8] * pT[8:9])
    sel = i1 > i0
    iou_best = jnp.maximum(i0, i1)

    conf = tT[4:5]
    coord = (conf == 1.0).astype(jnp.float32)
    noobj = (conf == 0.0).astype(jnp.float32)

    dxy = pT[0:2] - tT[0:2]
    dxy1 = pT[5:7] - tT[5:7]
    d2xy = dxy * dxy
    d2xy1 = dxy1 * dxy1
    xy_row = jnp.where(sel, d2xy1[0:1] + d2xy1[1:2], d2xy[0:1] + d2xy[1:2])

    swh = jnp.sqrt(pT[2:4]) - jnp.sqrt(tT[2:4])
    swh1 = jnp.sqrt(pT[7:9]) - jnp.sqrt(tT[7:9])
    s2 = swh * swh
    s21 = swh1 * swh1
    wh_row = jnp.where(sel, s21[0:1] + s21[1:2], s2[0:1] + s2[1:2])

    cp = jnp.where(sel, pT[9:10], pT[4:5])
    obj_row = (cp - iou_best) ** 2

    dc0 = pT[4:5] - tT[4:5]
    dc1 = pT[9:10] - tT[9:10]
    noobj_row = dc0 * dc0 + dc1 * dc1

    dcl = pT[10:30] - tT[10:30]
    class_row = jnp.sum(dcl * dcl, axis=0, keepdims=True)

    per_row = coord * (LOBJ * (xy_row + wh_row) + obj_row + class_row) \
        + LNOBJ * noobj * noobj_row
    return jnp.sum(per_row, axis=(0, 1), keepdims=True)


def _kernel_body(p_ref, t_ref, out_ref):
    @pl.when(pl.program_id(0) == 0)
    def _init():
        out_ref[...] = jnp.zeros_like(out_ref)

    pT = p_ref[...].T
    tT = t_ref[...].T
    total = None
    for s in range(CELLS_PER_ROW):
        part = _loss_slab(pT[N * s:N * (s + 1)], tT[N * s:N * (s + 1)])
        total = part if total is None else total + part
    out_ref[...] += total


def kernel(P, T):
    batch = P.shape[0]
    Pf = P.reshape(-1, LANES)
    Tf = T.reshape(-1, LANES)
    rows = Pf.shape[0]
    r = ROWS_PER_BLOCK
    grid = rows // r

    out = pl.pallas_call(
        _kernel_body,
        grid=(grid,),
        in_specs=[
            pl.BlockSpec((r, LANES), lambda i: (i, 0)),
            pl.BlockSpec((r, LANES), lambda i: (i, 0)),
        ],
        out_specs=pl.BlockSpec((1, 1), lambda i: (0, 0)),
        out_shape=jax.ShapeDtypeStruct((1, 1), jnp.float32),
        compiler_params=pltpu.CompilerParams(
            dimension_semantics=("arbitrary",),
        ),
    )(Pf, Tf)
    return out[0, 0] / batch
